# baseline (speedup 1.0000x reference)
import functools

import jax
import jax.numpy as jnp
from jax import lax
from jax.experimental import pallas as pl
from jax.experimental.pallas import tpu as pltpu
from jax.experimental.pallas import tpu_sc as plsc

_MAXG = 128


@functools.lru_cache(maxsize=None)
def _make_sc_gather(n_rows: int, n_atoms: int, n_per_atom: int, embed_dim: int):
    info = plsc.get_sparse_core_info()
    nw = info.num_cores * info.num_subcores
    assert n_atoms % (8 * nw) == 0 and n_per_atom % 8 == 0
    atoms_per_w = n_atoms // nw
    n_lines = n_atoms // 2 * n_per_atom
    segs = []
    off = 0
    while off < n_per_atom:
        n = min(_MAXG, n_per_atom - off)
        segs.append((off, n))
        off += n

    mesh = plsc.VectorSubcoreMesh(core_axis_name="c", subcore_axis_name="s")

    @functools.partial(
        pl.kernel,
        mesh=mesh,
        out_type=jax.ShapeDtypeStruct((n_lines, 2 * embed_dim), jnp.float32),
        scratch_types=[
            pltpu.VMEM((atoms_per_w, n_per_atom), jnp.int32),
            pltpu.VMEM((n_per_atom, embed_dim), jnp.float32),
            pltpu.VMEM((n_per_atom, embed_dim), jnp.float32),
            pltpu.VMEM_SHARED((n_rows, embed_dim), jnp.float32),
            pltpu.SemaphoreType.DMA,
            pltpu.SemaphoreType.DMA,
            pltpu.SemaphoreType.DMA,
            pltpu.SemaphoreType.DMA,
        ],
        compiler_params=pltpu.CompilerParams(use_tc_tiling_on_sc=False),
    )
    def k(table_hbm, idx_hbm, y_hbm, idx_v, st0, st1, table_sp, g0, g1, s0, s1):
        bufs = ((st0, g0, s0), (st1, g1, s1))
        sid = lax.axis_index("s")
        wid = sid * info.num_cores + lax.axis_index("c")
        a_base = wid * atoms_per_w

        @pl.when(sid == 0)
        def _():
            pltpu.sync_copy(table_hbm, table_sp)

        pltpu.sync_copy(idx_hbm.at[pl.ds(a_base, atoms_per_w)], idx_v)
        plsc.subcore_barrier()

        def gathers(a, p):
            st, sg, _ = bufs[p]
            return [
                pltpu.make_async_copy(
                    table_sp.at[idx_v.at[a, pl.ds(off, n)]],
                    st.at[pl.ds(off, n)],
                    sg,
                )
                for off, n in segs
            ]

        def scat(a, p):
            st, _, sc = bufs[p]
            ga = a_base + a
            blk = ga // 8
            pos = ga % 8
            line0 = (blk * 4 + pos % 4) * n_per_atom
            col0 = (pos // 4) * embed_dim
            return [
                pltpu.make_async_copy(
                    st,
                    y_hbm.at[
                        pl.ds(line0, n_per_atom), pl.ds(col0, embed_dim)
                    ],
                    sc,
                )
            ]

        def start(cs):
            for c in cs:
                c.start()

        def wait(cs):
            for c in cs:
                c.wait()

        start(gathers(0, 0))
        start(gathers(1, 1))
        wait(gathers(0, 0))
        start(scat(0, 0))
        wait(gathers(1, 1))
        start(scat(1, 1))

        def body(i, _):
            a0 = 2 * i
            wait(scat(a0 - 2, 0))
            start(gathers(a0, 0))
            wait(scat(a0 - 1, 1))
            start(gathers(a0 + 1, 1))
            wait(gathers(a0, 0))
            start(scat(a0, 0))
            wait(gathers(a0 + 1, 1))
            start(scat(a0 + 1, 1))
            return 0

        lax.fori_loop(1, atoms_per_w // 2, body, 0)
        wait(scat(atoms_per_w - 2, 0))
        wait(scat(atoms_per_w - 1, 1))

    return k


@functools.lru_cache(maxsize=None)
def _make_tc_unpack(n_atoms: int, n_per_atom: int, embed_dim: int):
    blk_atoms = 8
    assert n_atoms % blk_atoms == 0
    lines_per_blk = (blk_atoms // 2) * n_per_atom

    def body(y_ref, o_ref):
        y3 = y_ref[...].reshape(blk_atoms // 2, n_per_atom, 2 * embed_dim)
        o_ref[: blk_atoms // 2] = y3[:, :, :embed_dim]
        o_ref[blk_atoms // 2 :] = y3[:, :, embed_dim:]

    return pl.pallas_call(
        body,
        grid=(n_atoms // blk_atoms,),
        in_specs=[pl.BlockSpec((lines_per_blk, 2 * embed_dim), lambda i: (i, 0))],
        out_specs=pl.BlockSpec(
            (blk_atoms, n_per_atom, embed_dim), lambda i: (i, 0, 0)
        ),
        out_shape=jax.ShapeDtypeStruct(
            (n_atoms, n_per_atom, embed_dim), jnp.float32
        ),
    )


def kernel(atype, table):
    b0, b1 = atype.shape
    n_rows, embed_dim = table.shape
    idx = atype.astype(jnp.int32)
    y = _make_sc_gather(n_rows, b0, b1, embed_dim)(table, idx)
    return _make_tc_unpack(b0, b1, embed_dim)(y)
